# baseline (speedup 1.0000x reference)
import jax
import jax.numpy as jnp
from jax.experimental import pallas as pl
from jax.experimental.pallas import tpu as pltpu

_EPS = 1e-5
_QMAX = 127.0
_R = 128


def _fused_body(x1_ref, x2_ref, gamma_ref, ss1_ref, ss2_ref,
                xsum_ref, ynorm_ref, y1_ref, s1_ref, y2_ref, s2_ref):
    xs = x1_ref[...] + x2_ref[...]
    xsum_ref[...] = xs
    ms = jnp.mean(xs * xs, axis=-1, keepdims=True)
    inv_rms = jax.lax.rsqrt(ms + _EPS)
    yn = xs * inv_rms * gamma_ref[...]
    ynorm_ref[...] = yn
    for ss_ref, y_ref, s_ref in ((ss1_ref, y1_ref, s1_ref),
                                 (ss2_ref, y2_ref, s2_ref)):
        ys = yn * ss_ref[...]
        m = jnp.max(jnp.abs(ys), axis=-1, keepdims=True)
        s_ref[...] = (jnp.transpose(m, (1, 0)) * (1.0 / _QMAX))[None]
        yq = jnp.round(ys * (_QMAX / m))
        y_ref[...] = jnp.clip(yq, -128.0, 127.0).astype(jnp.int8)


def kernel(x1, x2, gamma, smooth_scale1, smooth_scale2):
    B, S, N = x1.shape
    rows = B * S
    nblk = rows // _R
    grid = (nblk,)

    x1f = x1.reshape(rows, N)
    x2f = x2.reshape(rows, N)
    g2 = gamma.reshape(1, N)
    ss1 = smooth_scale1.reshape(1, N)
    ss2 = smooth_scale2.reshape(1, N)

    row_spec = pl.BlockSpec((_R, N), lambda i: (i, 0))
    vec_spec = pl.BlockSpec((1, N), lambda i: (0, 0))
    scl_spec = pl.BlockSpec((1, 1, _R), lambda i: (i, 0, 0))

    f32 = jnp.float32
    outs = pl.pallas_call(
        _fused_body,
        grid=grid,
        in_specs=[row_spec, row_spec, vec_spec, vec_spec, vec_spec],
        out_specs=[row_spec, row_spec, row_spec, scl_spec, row_spec, scl_spec],
        out_shape=[
            jax.ShapeDtypeStruct((rows, N), f32),
            jax.ShapeDtypeStruct((rows, N), f32),
            jax.ShapeDtypeStruct((rows, N), jnp.int8),
            jax.ShapeDtypeStruct((nblk, 1, _R), f32),
            jax.ShapeDtypeStruct((rows, N), jnp.int8),
            jax.ShapeDtypeStruct((nblk, 1, _R), f32),
        ],
        compiler_params=pltpu.CompilerParams(
            dimension_semantics=("parallel",),
            vmem_limit_bytes=100 * 1024 * 1024,
        ),
    )(x1f, x2f, g2, ss1, ss2)

    xsum, ynorm, y1, s1, y2, s2 = outs
    return (xsum.reshape(B, S, N), ynorm.reshape(B, S, N),
            y1.reshape(B, S, N), s1.reshape(B, S),
            y2.reshape(B, S, N), s2.reshape(B, S))


# speedup vs baseline: 1.0337x; 1.0337x over previous
import jax
import jax.numpy as jnp
from jax.experimental import pallas as pl
from jax.experimental.pallas import tpu as pltpu

_EPS = 1e-5
_QMAX = 127.0
_R = 256


def _fused_body(x1_ref, x2_ref, gamma_ref, ss1_ref, ss2_ref,
                xsum_ref, ynorm_ref, y1_ref, s1_ref, y2_ref, s2_ref):
    pid = pl.program_id(0)
    xs = x1_ref[...] + x2_ref[...]
    xsum_ref[...] = xs
    ms = jnp.mean(xs * xs, axis=-1, keepdims=True)
    inv_rms = jax.lax.rsqrt(ms + _EPS)
    yn = xs * inv_rms * gamma_ref[...]
    ynorm_ref[...] = yn
    for ss_ref, y_ref, s_ref in ((ss1_ref, y1_ref, s1_ref),
                                 (ss2_ref, y2_ref, s2_ref)):
        ys = yn * ss_ref[...]
        m = jnp.max(jnp.abs(ys), axis=-1, keepdims=True)
        s_ref[pl.ds(pid, 1)] = (jnp.transpose(m, (1, 0)) * (1.0 / _QMAX))[None]
        yq = jnp.round(ys * (_QMAX / m))
        y_ref[...] = jnp.clip(yq, -128.0, 127.0).astype(jnp.int8)


def kernel(x1, x2, gamma, smooth_scale1, smooth_scale2):
    B, S, N = x1.shape
    rows = B * S
    nblk = rows // _R
    grid = (nblk,)

    x1f = x1.reshape(rows, N)
    x2f = x2.reshape(rows, N)
    g2 = gamma.reshape(1, N)
    ss1 = smooth_scale1.reshape(1, N)
    ss2 = smooth_scale2.reshape(1, N)

    row_spec = pl.BlockSpec((_R, N), lambda i: (i, 0))
    vec_spec = pl.BlockSpec((1, N), lambda i: (0, 0))
    scl_spec = pl.BlockSpec((nblk, 1, _R), lambda i: (0, 0, 0))

    f32 = jnp.float32
    outs = pl.pallas_call(
        _fused_body,
        grid=grid,
        in_specs=[row_spec, row_spec, vec_spec, vec_spec, vec_spec],
        out_specs=[row_spec, row_spec, row_spec, scl_spec, row_spec, scl_spec],
        out_shape=[
            jax.ShapeDtypeStruct((rows, N), f32),
            jax.ShapeDtypeStruct((rows, N), f32),
            jax.ShapeDtypeStruct((rows, N), jnp.int8),
            jax.ShapeDtypeStruct((nblk, 1, _R), f32),
            jax.ShapeDtypeStruct((rows, N), jnp.int8),
            jax.ShapeDtypeStruct((nblk, 1, _R), f32),
        ],
        compiler_params=pltpu.CompilerParams(
            dimension_semantics=("arbitrary",),
            vmem_limit_bytes=100 * 1024 * 1024,
        ),
    )(x1f, x2f, g2, ss1, ss2)

    xsum, ynorm, y1, s1, y2, s2 = outs
    return (xsum.reshape(B, S, N), ynorm.reshape(B, S, N),
            y1.reshape(B, S, N), s1.reshape(B, S),
            y2.reshape(B, S, N), s2.reshape(B, S))


# speedup vs baseline: 1.0356x; 1.0018x over previous
import jax
import jax.numpy as jnp
from jax.experimental import pallas as pl
from jax.experimental.pallas import tpu as pltpu

_EPS = 1e-5
_QMAX = 127.0
_R = 256


def _fused_body(x1_ref, x2_ref, gamma_ref, ss1_ref, ss2_ref,
                xsum_ref, ynorm_ref, y1_ref, s1_ref, y2_ref, s2_ref):
    xs = x1_ref[...] + x2_ref[...]
    xsum_ref[...] = xs
    ms = jnp.mean(xs * xs, axis=-1, keepdims=True)
    inv_rms = jax.lax.rsqrt(ms + _EPS)
    yn = xs * inv_rms * gamma_ref[...]
    ynorm_ref[...] = yn
    for ss_ref, y_ref, s_ref in ((ss1_ref, y1_ref, s1_ref),
                                 (ss2_ref, y2_ref, s2_ref)):
        ys = yn * ss_ref[...]
        m = jnp.max(jnp.abs(ys), axis=-1, keepdims=True)
        s_ref[...] = (jnp.transpose(m, (1, 0)) * (1.0 / _QMAX))[None]
        yq = jnp.round(ys * (_QMAX / m))
        y_ref[...] = jnp.clip(yq, -128.0, 127.0).astype(jnp.int8)


def kernel(x1, x2, gamma, smooth_scale1, smooth_scale2):
    B, S, N = x1.shape
    rows = B * S
    nblk = rows // _R
    grid = (nblk,)

    x1f = x1.reshape(rows, N)
    x2f = x2.reshape(rows, N)
    g2 = gamma.reshape(1, N)
    ss1 = smooth_scale1.reshape(1, N)
    ss2 = smooth_scale2.reshape(1, N)

    row_spec = pl.BlockSpec((_R, N), lambda i: (i, 0))
    vec_spec = pl.BlockSpec((1, N), lambda i: (0, 0))
    scl_spec = pl.BlockSpec((1, 1, _R), lambda i: (i, 0, 0))

    f32 = jnp.float32
    outs = pl.pallas_call(
        _fused_body,
        grid=grid,
        in_specs=[row_spec, row_spec, vec_spec, vec_spec, vec_spec],
        out_specs=[row_spec, row_spec, row_spec, scl_spec, row_spec, scl_spec],
        out_shape=[
            jax.ShapeDtypeStruct((rows, N), f32),
            jax.ShapeDtypeStruct((rows, N), f32),
            jax.ShapeDtypeStruct((rows, N), jnp.int8),
            jax.ShapeDtypeStruct((nblk, 1, _R), f32),
            jax.ShapeDtypeStruct((rows, N), jnp.int8),
            jax.ShapeDtypeStruct((nblk, 1, _R), f32),
        ],
        compiler_params=pltpu.CompilerParams(
            dimension_semantics=("parallel",),
            vmem_limit_bytes=100 * 1024 * 1024,
        ),
    )(x1f, x2f, g2, ss1, ss2)

    xsum, ynorm, y1, s1, y2, s2 = outs
    return (xsum.reshape(B, S, N), ynorm.reshape(B, S, N),
            y1.reshape(B, S, N), s1.reshape(B, S),
            y2.reshape(B, S, N), s2.reshape(B, S))
